# Initial kernel scaffold; baseline (speedup 1.0000x reference)
import jax
import jax.numpy as jnp
from jax.experimental import pallas as pl


def kernel(embedding, neighbor_idx, self_feats):
    raise NotImplementedError("write your pallas kernel here")


# speedup vs baseline: 4.0329x; 4.0329x over previous
import functools

import jax
import jax.numpy as jnp
from jax import lax
from jax.experimental import pallas as pl
from jax.experimental.pallas import tpu as pltpu
from jax.experimental.pallas import tpu_sc as plsc

N_EMBED = 10000
B = 4096
K = 32
D = 512
L = 16

NW = 32
BPW = B // NW
G = 2
RPG = G * K
GROUPS = BPW // G
BLK = 16
NBLK = BPW // BLK
GPB = BLK // G


def _make_kernel():
  mesh = plsc.VectorSubcoreMesh(core_axis_name="c", subcore_axis_name="s")

  @functools.partial(
      pl.kernel,
      out_type=jax.ShapeDtypeStruct((B, 2 * D), jnp.float32),
      mesh=mesh,
      scratch_types=[
          pltpu.VMEM((BPW * K,), jnp.int32),
          pltpu.VMEM((2, RPG, D), jnp.float32),
          pltpu.VMEM((2, BLK, D), jnp.float32),
          pltpu.VMEM((2, BLK, 2 * D), jnp.float32),
          pltpu.SemaphoreType.DMA,
          pltpu.SemaphoreType.DMA,
          pltpu.SemaphoreType.DMA,
          pltpu.SemaphoreType.DMA,
          pltpu.SemaphoreType.DMA,
          pltpu.SemaphoreType.DMA,
      ],
  )
  def agg(emb_hbm, idx_hbm, self_hbm, out_hbm,
          idx_v, rows_v, self_v, out_v, g0, g1, s0, s1, o0, o1):
    gsem = (g0, g1)
    ssem = (s0, s1)
    osem = (o0, o1)
    wid = lax.axis_index("s") * 2 + lax.axis_index("c")
    base = wid * BPW

    pltpu.sync_copy(idx_hbm.at[pl.ds(wid * (BPW * K), BPW * K)], idx_v)

    def start_gather(gi, p):
      pltpu.async_copy(
          emb_hbm.at[idx_v.at[pl.ds(gi * RPG, RPG)]], rows_v.at[p], gsem[p])

    def wait_gather(p):
      pltpu.make_async_copy(
          emb_hbm.at[pl.ds(0, RPG)], rows_v.at[p], gsem[p]).wait()

    def start_self(t, pb):
      pltpu.async_copy(
          self_hbm.at[pl.ds(base + t * BLK, BLK)], self_v.at[pb], ssem[pb])

    def wait_self(pb):
      pltpu.make_async_copy(
          self_hbm.at[pl.ds(0, BLK)], self_v.at[pb], ssem[pb]).wait()

    def start_out(t, pb):
      pltpu.async_copy(
          out_v.at[pb], out_hbm.at[pl.ds(base + t * BLK, BLK)], osem[pb])

    def wait_out(pb):
      pltpu.make_async_copy(
          out_v.at[pb], out_hbm.at[pl.ds(0, BLK)], osem[pb]).wait()

    start_gather(0, 0)
    start_self(0, 0)

    def do_group(gi, gg, p, pb):
      @pl.when(gi + 1 < GROUPS)
      def _():
        start_gather(gi + 1, 1 - p)

      wait_gather(p)

      for d in range(G):
        r0 = d * K
        row = gg * G + d

        @pl.loop(0, D // L)
        def _(ci):
          off = ci * L
          a0 = rows_v[p, r0 + 0, pl.ds(off, L)]
          a1 = rows_v[p, r0 + 1, pl.ds(off, L)]
          a2 = rows_v[p, r0 + 2, pl.ds(off, L)]
          a3 = rows_v[p, r0 + 3, pl.ds(off, L)]
          for k in range(4, K, 4):
            a0 = a0 + rows_v[p, r0 + k + 0, pl.ds(off, L)]
            a1 = a1 + rows_v[p, r0 + k + 1, pl.ds(off, L)]
            a2 = a2 + rows_v[p, r0 + k + 2, pl.ds(off, L)]
            a3 = a3 + rows_v[p, r0 + k + 3, pl.ds(off, L)]
          m = ((a0 + a1) + (a2 + a3)) * (1.0 / K)
          out_v[pb, row, pl.ds(off, L)] = m
          out_v[pb, row, pl.ds(D + off, L)] = (
              self_v[pb, row, pl.ds(off, L)] - m)

    def do_block(t, pb):
      @pl.when(t + 1 < NBLK)
      def _():
        start_self(t + 1, 1 - pb)

      wait_self(pb)

      @pl.when(t >= 2)
      def _():
        wait_out(pb)

      @pl.loop(0, GPB // 2)
      def _(gh):
        for p in range(2):
          gg = gh * 2 + p
          do_group(t * GPB + gg, gg, p, pb)

      start_out(t, pb)

    @pl.loop(0, NBLK // 2)
    def _(th):
      do_block(th * 2, 0)
      do_block(th * 2 + 1, 1)

    wait_out(0)
    wait_out(1)

  return agg


_agg = jax.jit(_make_kernel())


@jax.jit
def kernel(embedding, neighbor_idx, self_feats):
  idx_flat = neighbor_idx.reshape(-1)
  return _agg(embedding, idx_flat, self_feats)


# speedup vs baseline: 4.5347x; 1.1244x over previous
import functools

import jax
import jax.numpy as jnp
from jax import lax
from jax.experimental import pallas as pl
from jax.experimental.pallas import tpu as pltpu
from jax.experimental.pallas import tpu_sc as plsc

N_EMBED = 10000
B = 4096
K = 32
D = 512
L = 16

NW = 32
BPW = B // NW
G = 2
RPG = G * K
GROUPS = BPW // G
BLK = 16
NBLK = BPW // BLK
GPB = BLK // G


def _make_kernel():
  mesh = plsc.VectorSubcoreMesh(core_axis_name="c", subcore_axis_name="s")

  @functools.partial(
      pl.kernel,
      out_type=jax.ShapeDtypeStruct((B, 2 * D), jnp.float32),
      mesh=mesh,
      scratch_types=[
          pltpu.VMEM((BPW * K,), jnp.int32),
          pltpu.VMEM((2, RPG, D), jnp.float32),
          pltpu.VMEM((2, BLK, D), jnp.float32),
          pltpu.VMEM((2, BLK, 2 * D), jnp.float32),
          pltpu.SemaphoreType.DMA,
          pltpu.SemaphoreType.DMA,
          pltpu.SemaphoreType.DMA,
          pltpu.SemaphoreType.DMA,
          pltpu.SemaphoreType.DMA,
          pltpu.SemaphoreType.DMA,
      ],
  )
  def agg(emb_hbm, idx_hbm, self_hbm, out_hbm,
          idx_v, rows_v, self_v, out_v, g0, g1, s0, s1, o0, o1):
    gsem = (g0, g1)
    ssem = (s0, s1)
    osem = (o0, o1)
    wid = lax.axis_index("s") * 2 + lax.axis_index("c")
    base = wid * BPW

    pltpu.sync_copy(idx_hbm.at[pl.ds(wid * (BPW * K), BPW * K)], idx_v)

    def start_gather(gi, p):
      pltpu.async_copy(
          emb_hbm.at[idx_v.at[pl.ds(gi * RPG, RPG)]], rows_v.at[p], gsem[p])

    def wait_gather(p):
      pltpu.make_async_copy(
          emb_hbm.at[pl.ds(0, RPG)], rows_v.at[p], gsem[p]).wait()

    def start_self(t, pb):
      pltpu.async_copy(
          self_hbm.at[pl.ds(base + t * BLK, BLK)], self_v.at[pb], ssem[pb])

    def wait_self(pb):
      pltpu.make_async_copy(
          self_hbm.at[pl.ds(0, BLK)], self_v.at[pb], ssem[pb]).wait()

    def start_out(t, pb):
      pltpu.async_copy(
          out_v.at[pb], out_hbm.at[pl.ds(base + t * BLK, BLK)], osem[pb])

    def wait_out(pb):
      pltpu.make_async_copy(
          out_v.at[pb], out_hbm.at[pl.ds(0, BLK)], osem[pb]).wait()

    start_gather(0, 0)
    start_self(0, 0)

    def do_group(gi, gg, p, pb):
      @pl.when(gi + 1 < GROUPS)
      def _():
        start_gather(gi + 1, 1 - p)

      wait_gather(p)

      for d in range(G):
        r0 = d * K
        row = gg * G + d

        @plsc.parallel_loop(0, D // L, unroll=2)
        def _(ci):
          off = ci * L
          a0 = rows_v[p, r0 + 0, pl.ds(off, L)]
          a1 = rows_v[p, r0 + 1, pl.ds(off, L)]
          a2 = rows_v[p, r0 + 2, pl.ds(off, L)]
          a3 = rows_v[p, r0 + 3, pl.ds(off, L)]
          for k in range(4, K, 4):
            a0 = a0 + rows_v[p, r0 + k + 0, pl.ds(off, L)]
            a1 = a1 + rows_v[p, r0 + k + 1, pl.ds(off, L)]
            a2 = a2 + rows_v[p, r0 + k + 2, pl.ds(off, L)]
            a3 = a3 + rows_v[p, r0 + k + 3, pl.ds(off, L)]
          m = ((a0 + a1) + (a2 + a3)) * (1.0 / K)
          out_v[pb, row, pl.ds(off, L)] = m
          out_v[pb, row, pl.ds(D + off, L)] = (
              self_v[pb, row, pl.ds(off, L)] - m)

    def do_block(t, pb):
      @pl.when(t + 1 < NBLK)
      def _():
        start_self(t + 1, 1 - pb)

      wait_self(pb)

      @pl.when(t >= 2)
      def _():
        wait_out(pb)

      @pl.loop(0, GPB // 2)
      def _(gh):
        for p in range(2):
          gg = gh * 2 + p
          do_group(t * GPB + gg, gg, p, pb)

      start_out(t, pb)

    @pl.loop(0, NBLK // 2)
    def _(th):
      do_block(th * 2, 0)
      do_block(th * 2 + 1, 1)

    wait_out(0)
    wait_out(1)

  return agg


_agg = jax.jit(_make_kernel())


@jax.jit
def kernel(embedding, neighbor_idx, self_feats):
  idx_flat = neighbor_idx.reshape(-1)
  return _agg(embedding, idx_flat, self_feats)


# speedup vs baseline: 4.7327x; 1.0437x over previous
import functools

import jax
import jax.numpy as jnp
from jax import lax
from jax.experimental import pallas as pl
from jax.experimental.pallas import tpu as pltpu
from jax.experimental.pallas import tpu_sc as plsc

N_EMBED = 10000
B = 4096
K = 32
D = 512
L = 16

NW = 32
BPW = B // NW
NBUF = 4
BLK = 16
NBLK = BPW // BLK


def _make_kernel():
  mesh = plsc.VectorSubcoreMesh(core_axis_name="c", subcore_axis_name="s")

  @functools.partial(
      pl.kernel,
      out_type=jax.ShapeDtypeStruct((B, 2 * D), jnp.float32),
      mesh=mesh,
      scratch_types=[
          pltpu.VMEM((BPW, K), jnp.int32),
          pltpu.VMEM((NBUF, K, D), jnp.float32),
          pltpu.VMEM((2, BLK, D), jnp.float32),
          pltpu.VMEM((2, BLK, 2 * D), jnp.float32),
          pltpu.SemaphoreType.DMA,
          pltpu.SemaphoreType.DMA,
          pltpu.SemaphoreType.DMA,
          pltpu.SemaphoreType.DMA,
          pltpu.SemaphoreType.DMA,
          pltpu.SemaphoreType.DMA,
          pltpu.SemaphoreType.DMA,
          pltpu.SemaphoreType.DMA,
      ],
  )
  def agg(emb_hbm, idx_hbm, self_hbm, out_hbm,
          idx_v, rows_v, self_v, out_v, g0, g1, g2, g3, s0, s1, o0, o1):
    gsem = (g0, g1, g2, g3)
    ssem = (s0, s1)
    osem = (o0, o1)
    wid = lax.axis_index("s") * 2 + lax.axis_index("c")
    base = wid * BPW

    pltpu.sync_copy(idx_hbm.at[pl.ds(base, BPW)], idx_v)

    def start_gather(g, p):
      pltpu.async_copy(emb_hbm.at[idx_v.at[g]], rows_v.at[p], gsem[p])

    def wait_gather(p):
      pltpu.make_async_copy(
          emb_hbm.at[pl.ds(0, K)], rows_v.at[p], gsem[p]).wait()

    def start_self(t, pb):
      pltpu.async_copy(
          self_hbm.at[pl.ds(base + t * BLK, BLK)], self_v.at[pb], ssem[pb])

    def wait_self(pb):
      pltpu.make_async_copy(
          self_hbm.at[pl.ds(0, BLK)], self_v.at[pb], ssem[pb]).wait()

    def start_out(t, pb):
      pltpu.async_copy(
          out_v.at[pb], out_hbm.at[pl.ds(base + t * BLK, BLK)], osem[pb])

    def wait_out(pb):
      pltpu.make_async_copy(
          out_v.at[pb], out_hbm.at[pl.ds(0, BLK)], osem[pb]).wait()

    for p in range(NBUF - 1):
      start_gather(p, p)
    start_self(0, 0)

    def do_group(g, gg, p, pb):
      @pl.when(g + (NBUF - 1) < BPW)
      def _():
        start_gather(g + (NBUF - 1), (p + (NBUF - 1)) % NBUF)

      wait_gather(p)

      @plsc.parallel_loop(0, D // L, unroll=2)
      def _(ci):
        off = ci * L
        a0 = rows_v[p, 0, pl.ds(off, L)]
        a1 = rows_v[p, 1, pl.ds(off, L)]
        a2 = rows_v[p, 2, pl.ds(off, L)]
        a3 = rows_v[p, 3, pl.ds(off, L)]
        for k in range(4, K, 4):
          a0 = a0 + rows_v[p, k + 0, pl.ds(off, L)]
          a1 = a1 + rows_v[p, k + 1, pl.ds(off, L)]
          a2 = a2 + rows_v[p, k + 2, pl.ds(off, L)]
          a3 = a3 + rows_v[p, k + 3, pl.ds(off, L)]
        m = ((a0 + a1) + (a2 + a3)) * (1.0 / K)
        out_v[pb, gg, pl.ds(off, L)] = m
        out_v[pb, gg, pl.ds(D + off, L)] = self_v[pb, gg, pl.ds(off, L)] - m

    def do_block(t, pb):
      @pl.when(t + 1 < NBLK)
      def _():
        start_self(t + 1, 1 - pb)

      wait_self(pb)

      @pl.when(t >= 2)
      def _():
        wait_out(pb)

      for gg in range(BLK):
        do_group(t * BLK + gg, gg, gg % NBUF, pb)

      start_out(t, pb)

    @pl.loop(0, NBLK // 2)
    def _(th):
      do_block(th * 2, 0)
      do_block(th * 2 + 1, 1)

    wait_out(0)
    wait_out(1)

  return agg


_agg = jax.jit(_make_kernel())


@jax.jit
def kernel(embedding, neighbor_idx, self_feats):
  return _agg(embedding, neighbor_idx, self_feats)
